# Initial kernel scaffold; baseline (speedup 1.0000x reference)
import jax
import jax.numpy as jnp
from jax.experimental import pallas as pl


def kernel(neighbor_features, transaction_features, edge_index, neighbor_id, W1, b1, W2, b2, Wf1, bf1, Wf2, bf2):
    raise NotImplementedError("write your pallas kernel here")


# speedup vs baseline: 12.4371x; 12.4371x over previous
import functools

import jax
import jax.numpy as jnp
from jax import lax
from jax.experimental import pallas as pl
from jax.experimental.pallas import tpu as pltpu
from jax.experimental.pallas import tpu_sc as plsc

N = 10000
E = 160000
CH = 128
EROWS = E // CH
NC, NS = 2, 16
F = 128
B = 4096

_sc_mesh = lambda: plsc.VectorSubcoreMesh(core_axis_name="c", subcore_axis_name="s")

SLAB = 640
SLAB_LAST = N - 15 * SLAB
EITERS = (EROWS + NS - 1) // NS


def _hist_body(dst_hbm, out_hbm, dstv, onesv, zbuf, acc):
    c = lax.axis_index("c")
    s = lax.axis_index("s")

    @pl.when(s == 0)
    def _():
        def zb(i, _):
            zbuf[pl.ds(i * 16, 16)] = jnp.zeros((16,), jnp.float32)
            return 0
        lax.fori_loop(0, N // 16, zb, 0)
        pltpu.sync_copy(zbuf, acc)

    def ob(i, _):
        onesv[pl.ds(i * 16, 16)] = jnp.ones((16,), jnp.float32)
        return 0
    lax.fori_loop(0, CH // 16, ob, 0)

    plsc.subcore_barrier()

    w = s * NC + c
    def body(k, _):
        j = w + k * (NC * NS)
        @pl.when(j < EROWS)
        def _():
            pltpu.sync_copy(dst_hbm.at[j], dstv)
            pltpu.sync_copy(onesv, acc.at[dstv], add=True)
        return 0
    lax.fori_loop(0, (EROWS + NC * NS - 1) // (NC * NS), body, 0)

    plsc.subcore_barrier()

    @pl.when(s == 0)
    def _():
        pltpu.sync_copy(acc, out_hbm.at[c])


def _sc_hist(dst2):
    return pl.kernel(
        _hist_body,
        out_type=jax.ShapeDtypeStruct((NC, N), jnp.float32),
        mesh=_sc_mesh(),
        scratch_types=[
            pltpu.VMEM((CH,), jnp.int32),
            pltpu.VMEM((CH,), jnp.float32),
            pltpu.VMEM((N,), jnp.float32),
            pltpu.VMEM_SHARED((N,), jnp.float32),
        ],
    )(dst2)


def _edge_body(with_gather, xl_hbm, xr_hbm, src_hbm, dst_hbm, *rest):
    if with_gather:
        (nid_hbm, dp_hbm, el_hbm, er_hbm, dn_hbm,
         srcv, dstv, rows, acc, nidv, embv, dnv) = rest
    else:
        (al_hbm, ar_hbm, srcv, dstv, rows, acc) = rest
    c = lax.axis_index("c")
    s = lax.axis_index("s")

    def init_from(x_hbm):
        @pl.when(s < NS - 1)
        def _():
            pltpu.sync_copy(x_hbm.at[pl.ds(s * SLAB, SLAB)],
                            acc.at[pl.ds(s * SLAB, SLAB)])
        @pl.when(s == NS - 1)
        def _():
            pltpu.sync_copy(x_hbm.at[pl.ds(15 * SLAB, SLAB_LAST)],
                            acc.at[pl.ds(15 * SLAB, SLAB_LAST)])

    def edge_loop(x_hbm):
        def body(k, _):
            j = s + k * NS
            @pl.when(j < EROWS)
            def _():
                pltpu.sync_copy(src_hbm.at[j], srcv)
                pltpu.sync_copy(dst_hbm.at[j], dstv)
                pltpu.sync_copy(x_hbm.at[srcv], rows)
                pltpu.sync_copy(rows, acc.at[dstv], add=True)
            return 0
        lax.fori_loop(0, EITERS, body, 0)

    @pl.when(c == 0)
    def _():
        init_from(xl_hbm)
    @pl.when(c == 1)
    def _():
        init_from(xr_hbm)
    plsc.subcore_barrier()
    @pl.when(c == 0)
    def _():
        edge_loop(xl_hbm)
    @pl.when(c == 1)
    def _():
        edge_loop(xr_hbm)
    plsc.subcore_barrier()

    if not with_gather:
        def writeout(out_hbm):
            @pl.when(s < NS - 1)
            def _():
                pltpu.sync_copy(acc.at[pl.ds(s * SLAB, SLAB)],
                                out_hbm.at[pl.ds(s * SLAB, SLAB)])
            @pl.when(s == NS - 1)
            def _():
                pltpu.sync_copy(acc.at[pl.ds(15 * SLAB, SLAB_LAST)],
                                out_hbm.at[pl.ds(15 * SLAB, SLAB_LAST)])
        @pl.when(c == 0)
        def _():
            writeout(al_hbm)
        @pl.when(c == 1)
        def _():
            writeout(ar_hbm)
    else:
        def emb_out(out_hbm):
            for t in range(B // CH // NS):
                j = s * (B // CH // NS) + t
                pltpu.sync_copy(nid_hbm.at[j], nidv)
                pltpu.sync_copy(acc.at[nidv], embv)
                pltpu.sync_copy(embv, out_hbm.at[pl.ds(j * CH, CH)])
                @pl.when(c == 0)
                def _():
                    pltpu.sync_copy(dp_hbm.at[nidv], dnv)
                    pltpu.sync_copy(dnv, dn_hbm.at[pl.ds(j * CH, CH)])
        @pl.when(c == 0)
        def _():
            emb_out(el_hbm)
        @pl.when(c == 1)
        def _():
            emb_out(er_hbm)


def _sc_edge(xl, xr, src2, dst2):
    return pl.kernel(
        functools.partial(_edge_body, False),
        out_type=(jax.ShapeDtypeStruct((N, F), jnp.float32),
                  jax.ShapeDtypeStruct((N, F), jnp.float32)),
        mesh=_sc_mesh(),
        scratch_types=[
            pltpu.VMEM((CH,), jnp.int32),
            pltpu.VMEM((CH,), jnp.int32),
            pltpu.VMEM((CH, F), jnp.float32),
            pltpu.VMEM_SHARED((N, F), jnp.float32),
        ],
    )(xl, xr, src2, dst2)


def _sc_edge_gather(xl, xr, src2, dst2, nid2, dispad):
    return pl.kernel(
        functools.partial(_edge_body, True),
        out_type=(jax.ShapeDtypeStruct((B, F), jnp.float32),
                  jax.ShapeDtypeStruct((B, F), jnp.float32),
                  jax.ShapeDtypeStruct((B, F), jnp.float32)),
        mesh=_sc_mesh(),
        scratch_types=[
            pltpu.VMEM((CH,), jnp.int32),
            pltpu.VMEM((CH,), jnp.int32),
            pltpu.VMEM((CH, F), jnp.float32),
            pltpu.VMEM_SHARED((N, F), jnp.float32),
            pltpu.VMEM((CH,), jnp.int32),
            pltpu.VMEM((CH, F), jnp.float32),
            pltpu.VMEM((CH, F), jnp.float32),
        ],
    )(xl, xr, src2, dst2, nid2, dispad)


def _prep_a_body(h_ref, d_ref):
    deg = 1.0 + h_ref[0] + h_ref[1]
    d_ref[...] = lax.rsqrt(deg)


def _tc_prep_a(hist):
    return pl.pallas_call(
        _prep_a_body,
        out_shape=jax.ShapeDtypeStruct((625, 16), jnp.float32),
    )(hist.reshape(NC, 625, 16))


RB = 2000


def _scale_split_body(x_ref, d_ref, xl_ref, xr_ref, dp_ref):
    dis = d_ref[...]
    xt = x_ref[...] * dis
    xl_ref[...] = xt[:, :F]
    xr_ref[...] = xt[:, F:]
    dp_ref[...] = jnp.broadcast_to(dis, (RB, F))


def _tc_scale_split(x, dis1):
    return pl.pallas_call(
        _scale_split_body,
        grid=(N // RB,),
        in_specs=[
            pl.BlockSpec((RB, 2 * F), lambda i: (i, 0)),
            pl.BlockSpec((RB, 1), lambda i: (i, 0)),
        ],
        out_specs=[
            pl.BlockSpec((RB, F), lambda i: (i, 0)),
            pl.BlockSpec((RB, F), lambda i: (i, 0)),
            pl.BlockSpec((RB, F), lambda i: (i, 0)),
        ],
        out_shape=[
            jax.ShapeDtypeStruct((N, F), jnp.float32),
            jax.ShapeDtypeStruct((N, F), jnp.float32),
            jax.ShapeDtypeStruct((N, F), jnp.float32),
        ],
    )(x, dis1)


def _mid_body(al_ref, ar_ref, d_ref, w1a_ref, w1b_ref, b1_ref, w2_ref,
              pl_ref, pr_ref):
    dis = d_ref[...]
    h = (jnp.dot(al_ref[...], w1a_ref[...], preferred_element_type=jnp.float32)
         + jnp.dot(ar_ref[...], w1b_ref[...], preferred_element_type=jnp.float32))
    h = jnp.maximum(dis * h + b1_ref[...], 0.0)
    p = jnp.dot(h, w2_ref[...], preferred_element_type=jnp.float32)
    pl_ref[...] = dis * p[:, :F]
    pr_ref[...] = dis * p[:, F:]


def _tc_mid(al, ar, dis1, w1a, w1b, b1, w2):
    return pl.pallas_call(
        _mid_body,
        grid=(N // RB,),
        in_specs=[
            pl.BlockSpec((RB, F), lambda i: (i, 0)),
            pl.BlockSpec((RB, F), lambda i: (i, 0)),
            pl.BlockSpec((RB, 1), lambda i: (i, 0)),
            pl.BlockSpec((F, 512), lambda i: (0, 0)),
            pl.BlockSpec((F, 512), lambda i: (0, 0)),
            pl.BlockSpec((1, 512), lambda i: (0, 0)),
            pl.BlockSpec((512, 2 * F), lambda i: (0, 0)),
        ],
        out_specs=[
            pl.BlockSpec((RB, F), lambda i: (i, 0)),
            pl.BlockSpec((RB, F), lambda i: (i, 0)),
        ],
        out_shape=[
            jax.ShapeDtypeStruct((N, F), jnp.float32),
            jax.ShapeDtypeStruct((N, F), jnp.float32),
        ],
    )(al, ar, dis1, w1a, w1b, b1, w2)


BB = 1024


def _mlp_body(el_ref, er_ref, dn_ref, tx_ref, b2_ref, wa_ref, wb_ref, wc_ref,
              bf1_ref, wf2_ref, bf2_ref, y_ref):
    dn = dn_ref[...][:, :1]
    e0 = dn * el_ref[...] + b2_ref[...][:, :F]
    e1 = dn * er_ref[...] + b2_ref[...][:, F:]
    h = (jnp.dot(e0, wa_ref[...], preferred_element_type=jnp.float32)
         + jnp.dot(e1, wb_ref[...], preferred_element_type=jnp.float32)
         + jnp.dot(tx_ref[...], wc_ref[...], preferred_element_type=jnp.float32))
    h = jnp.maximum(h + bf1_ref[...], 0.0)
    y_ref[...] = jnp.dot(h, wf2_ref[...], preferred_element_type=jnp.float32) + bf2_ref[...]


def _tc_mlp(el, er, disn, tx, b2, wa, wb, wc, bf1, wf2, bf2):
    return pl.pallas_call(
        _mlp_body,
        grid=(B // BB,),
        in_specs=[
            pl.BlockSpec((BB, F), lambda i: (i, 0)),
            pl.BlockSpec((BB, F), lambda i: (i, 0)),
            pl.BlockSpec((BB, F), lambda i: (i, 0)),
            pl.BlockSpec((BB, 64), lambda i: (i, 0)),
            pl.BlockSpec((1, 2 * F), lambda i: (0, 0)),
            pl.BlockSpec((F, 512), lambda i: (0, 0)),
            pl.BlockSpec((F, 512), lambda i: (0, 0)),
            pl.BlockSpec((64, 512), lambda i: (0, 0)),
            pl.BlockSpec((1, 512), lambda i: (0, 0)),
            pl.BlockSpec((512, 1), lambda i: (0, 0)),
            pl.BlockSpec((1, 1), lambda i: (0, 0)),
        ],
        out_specs=pl.BlockSpec((BB, 1), lambda i: (i, 0)),
        out_shape=jax.ShapeDtypeStruct((B, 1), jnp.float32),
    )(el, er, disn, tx, b2, wa, wb, wc, bf1, wf2, bf2)


def kernel(neighbor_features, transaction_features, edge_index, neighbor_id,
           W1, b1, W2, b2, Wf1, bf1, Wf2, bf2):
    src2 = edge_index[0].reshape(EROWS, CH)
    dst2 = edge_index[1].reshape(EROWS, CH)
    nid2 = neighbor_id.reshape(B // CH, CH)

    hist = _sc_hist(dst2)
    d625 = _tc_prep_a(hist)
    dis1 = d625.reshape(N, 1)
    xl, xr, dispad = _tc_scale_split(neighbor_features, dis1)
    al, ar = _sc_edge(xl, xr, src2, dst2)
    pl_, pr_ = _tc_mid(al, ar, dis1, W1[:F], W1[F:], b1.reshape(1, 512), W2)
    el, er, disn = _sc_edge_gather(pl_, pr_, src2, dst2, nid2, dispad)
    y = _tc_mlp(el, er, disn, transaction_features, b2.reshape(1, 2 * F),
                Wf1[:F], Wf1[F:2 * F], Wf1[2 * F:], bf1.reshape(1, 512),
                Wf2, bf2.reshape(1, 1))
    return y
